# baseline (speedup 1.0000x reference)
import functools

import jax
import jax.numpy as jnp
from jax import lax
from jax.experimental import pallas as pl
from jax.experimental.pallas import tpu as pltpu
from jax.experimental.pallas import tpu_sc as plsc

NC = 2
NS = 16
NW = NC * NS
C = 128
NBUF = 3


def _log_1to2(x):
    s = (x - 1.0) / (x + 1.0)
    s2 = s * s
    p = 1.0 / 9.0
    p = p * s2 + 1.0 / 7.0
    p = p * s2 + 1.0 / 5.0
    p = p * s2 + 1.0 / 3.0
    p = p * s2 + 1.0
    return (2.0 * s) * p


def _sqrt(x):
    i = lax.bitcast_convert_type(x, jnp.int32)
    y = lax.bitcast_convert_type(jnp.int32(0x5F3759DF) - (i >> 1), jnp.float32)
    for _ in range(3):
        y = y * (1.5 - 0.5 * x * y * y)
    return x * y


def _make_transpose(V, D):
    BLK = 256
    nfull = V // BLK
    rem = V - nfull * BLK
    base_cnt = nfull // NW
    extra = nfull - base_cnt * NW
    iters = -(-(base_cnt + (1 if extra else 0)) // 2) * 2
    assert iters >= 4 and rem % 16 == 0 and BLK % 16 == 0

    mesh = plsc.VectorSubcoreMesh(core_axis_name="c", subcore_axis_name="s")

    @functools.partial(
        pl.kernel,
        mesh=mesh,
        compiler_params=pltpu.CompilerParams(
            needs_layout_passes=False, use_tc_tiling_on_sc=True),
        out_type=jax.ShapeDtypeStruct((V // 2, 2 * D), jnp.float32),
        scratch_types=(
            [pltpu.VMEM((D, BLK), jnp.float32) for _ in range(2)]
            + [pltpu.VMEM((BLK // 2, 2 * D), jnp.float32) for _ in range(2)]
            + [pltpu.SemaphoreType.DMA for _ in range(4)]
        ),
    )
    def sc_transpose(wt_h, wtail_h, w2_h, *refs):
        inb = refs[0:2]
        outb = refs[2:4]
        semi = refs[4:6]
        semo = refs[6:8]

        wid = lax.axis_index("s") * NC + lax.axis_index("c")
        lanes = lax.iota(jnp.int32, 16)
        cnt = jnp.where(wid < extra, base_cnt + 1, base_cnt)
        bstart = wid * base_cnt + jnp.minimum(wid, extra)

        def blk_of(i):
            return bstart + jnp.minimum(i, cnt - 1)

        def fire_in(i, r):
            b = blk_of(i)
            for fb in range(D // 8):
                pltpu.async_copy(
                    wt_h.at[pl.ds(fb * 8, 8), pl.ds(b * BLK, BLK)],
                    inb[r].at[pl.ds(fb * 8, 8), :], semi[r])

        def wait_in(i, r):
            b = blk_of(i)
            for fb in range(D // 8):
                pltpu.make_async_copy(
                    wt_h.at[pl.ds(fb * 8, 8), pl.ds(b * BLK, BLK)],
                    inb[r].at[pl.ds(fb * 8, 8), :], semi[r]).wait()

        def fire_out(i, r):
            b = blk_of(i)
            pltpu.async_copy(
                outb[r], w2_h.at[pl.ds(b * (BLK // 2), BLK // 2)], semo[r])

        def wait_out(i, r):
            b = blk_of(i)
            pltpu.make_async_copy(
                outb[r], w2_h.at[pl.ds(b * (BLK // 2), BLK // 2)],
                semo[r]).wait()

        rowv = []
        colv = []
        for p in range(BLK // 16):
            ids = p * 16 + lanes
            rowv.append(ids >> 1)
            colv.append((ids & 1) * D)

        def transpose_blk(r):
            def fstep(f, _):
                for p in range(BLK // 16):
                    v = inb[r][f, pl.ds(p * 16, 16)]
                    plsc.store_scatter(outb[r], [rowv[p], colv[p] + f], v)
                return 0
            lax.fori_loop(0, D, fstep, 0)

        fire_in(0, 0)
        fire_in(1, 1)

        def step(k, _):
            for r in range(2):
                i = k * 2 + r

                @pl.when(i + 2 < iters)
                def _():
                    fire_in(i + 2, r)
                wait_in(i, r)

                @pl.when(i >= 2)
                def _():
                    wait_out(i - 2, r)
                transpose_blk(r)
                fire_out(i, r)
            return 0

        lax.fori_loop(0, iters // 2, step, 0)
        wait_out(iters - 2, 0)
        wait_out(iters - 1, 1)

        if rem:
            @pl.when(wid == NW - 1)
            def _():
                stage = outb[0].at[pl.ds(0, rem // 2)]
                pltpu.sync_copy(wtail_h, stage)
                pltpu.sync_copy(
                    stage, w2_h.at[pl.ds(nfull * (BLK // 2), rem // 2)])

    return sc_transpose


def kernel(pos_idxs, ys, neg_idxs, num_neg, W):
    B = pos_idxs.shape[1]
    NT = neg_idxs.shape[1]
    V, D = W.shape
    TOT = B + NT
    ppw = TOT // NW
    nch = ppw // C
    assert ppw % C == 0 and nch >= 2 * NBUF and D == 64 and V % 2 == 0
    scale = 1.0 / float(TOT)

    mesh = plsc.VectorSubcoreMesh(core_axis_name="c", subcore_axis_name="s")

    @functools.partial(
        pl.kernel,
        mesh=mesh,
        compiler_params=pltpu.CompilerParams(
            needs_layout_passes=False, use_tc_tiling_on_sc=True),
        out_type=jax.ShapeDtypeStruct((NW, 16), jnp.float32),
        scratch_types=(
            [pltpu.VMEM((ppw,), jnp.int32) for _ in range(4)]
            + [pltpu.VMEM((ppw,), jnp.float32)]
            + [pltpu.VMEM((C, 2 * D), jnp.float32) for _ in range(2 * NBUF)]
            + [pltpu.VMEM((16,), jnp.float32)]
            + [pltpu.SemaphoreType.DMA for _ in range(2 * NBUF)]
        ),
    )
    def sc_loss(i0_h, i1_h, yse_h, w2_h, out_h, *refs):
        rawu, rawv, fetu, fetv, ysa = refs[0:5]
        urows = refs[5:5 + NBUF]
        vrows = refs[5 + NBUF:5 + 2 * NBUF]
        accv = refs[5 + 2 * NBUF]
        semu = refs[6 + 2 * NBUF:6 + 2 * NBUF + NBUF]
        semv = refs[6 + 2 * NBUF + NBUF:6 + 2 * NBUF + 2 * NBUF]

        wid = lax.axis_index("s") * NC + lax.axis_index("c")
        tbase = wid * ppw
        lanes = lax.iota(jnp.int32, 16)

        pltpu.sync_copy(i0_h.at[pl.ds(tbase, ppw)], rawu)
        pltpu.sync_copy(i1_h.at[pl.ds(tbase, ppw)], rawv)
        pltpu.sync_copy(yse_h.at[pl.ds(tbase, ppw)], ysa)

        def shift_step(s, _):
            sl = pl.ds(s * 16, 16)
            fetu[sl] = rawu[sl] >> 1
            fetv[sl] = rawv[sl] >> 1
            return 0
        lax.fori_loop(0, ppw // 16, shift_step, 0)

        def fire(c, r):
            pltpu.async_copy(
                w2_h.at[fetu.at[pl.ds(c * C, C)]], urows[r], semu[r])
            pltpu.async_copy(
                w2_h.at[fetv.at[pl.ds(c * C, C)]], vrows[r], semv[r])

        def wait(c, r):
            pltpu.make_async_copy(
                w2_h.at[fetu.at[pl.ds(c * C, C)]], urows[r], semu[r]).wait()
            pltpu.make_async_copy(
                w2_h.at[fetv.at[pl.ds(c * C, C)]], vrows[r], semv[r]).wait()

        def compute(c, r, acc):
            def group(g, acc):
                rvec = g * 16 + lanes
                avec = c * C + rvec
                su = (plsc.load_gather(rawu, [avec]) & 1) * D
                sv = (plsc.load_gather(rawv, [avec]) & 1) * D
                dot = jnp.zeros((16,), jnp.float32)
                for j in range(D):
                    au = plsc.load_gather(urows[r], [rvec, su + j])
                    av = plsc.load_gather(vrows[r], [rvec, sv + j])
                    dot = dot + au * av
                z = jnp.where(tbase + avec < B, -dot, dot)
                t = jnp.exp(-jnp.abs(z))
                sp = jnp.maximum(z, 0.0) + _log_1to2(1.0 + t)
                yv = plsc.load_gather(ysa, [avec])
                alpha = _log_1to2(1.0 + _sqrt(yv)) + 1.0
                return acc + alpha * sp

            return lax.fori_loop(0, C // 16, group, acc)

        fire(0, 0)
        fire(1, 1)

        def step(k, acc):
            for r in range(NBUF):
                c = k * NBUF + r
                fire(c + 2, (r + 2) % NBUF)
                wait(c, r)
                acc = compute(c, r, acc)
            return acc

        assert nch % NBUF == 0
        acc = lax.fori_loop(0, nch // NBUF - 1, step,
                            jnp.zeros((16,), jnp.float32))
        c = nch - 3
        fire(nch - 1, (nch - 1) % NBUF)
        wait(c, c % NBUF)
        acc = compute(c, c % NBUF, acc)
        for c in range(nch - 2, nch):
            wait(c, c % NBUF)
            acc = compute(c, c % NBUF, acc)

        accv[...] = acc * scale
        pltpu.sync_copy(accv, out_h.at[wid])

    i0 = jnp.concatenate([pos_idxs[0], neg_idxs[0]])
    i1 = jnp.concatenate([pos_idxs[1], neg_idxs[1]])
    yse = jnp.concatenate([ys, jnp.zeros((NT,), jnp.float32)])
    rem = V % 256
    wtail = W[V - rem:].reshape(rem // 2, 2 * D) if rem else W[:8]
    w2 = _make_transpose(V, D)(W.T, wtail)
    partials = sc_loss(i0, i1, yse, w2)
    return jnp.sum(partials)


# speedup vs baseline: 1.1099x; 1.1099x over previous
import functools

import jax
import jax.numpy as jnp
from jax import lax
from jax.experimental import pallas as pl
from jax.experimental.pallas import tpu as pltpu
from jax.experimental.pallas import tpu_sc as plsc

NC = 2
NS = 16
NW = NC * NS
C = 128
NBUF = 3


def _log_1to2(x):
    s = (x - 1.0) / (x + 1.0)
    s2 = s * s
    p = 1.0 / 9.0
    p = p * s2 + 1.0 / 7.0
    p = p * s2 + 1.0 / 5.0
    p = p * s2 + 1.0 / 3.0
    p = p * s2 + 1.0
    return (2.0 * s) * p


def _sqrt(x):
    i = lax.bitcast_convert_type(x, jnp.int32)
    y = lax.bitcast_convert_type(jnp.int32(0x5F3759DF) - (i >> 1), jnp.float32)
    for _ in range(3):
        y = y * (1.5 - 0.5 * x * y * y)
    return x * y


def _make_transpose(V, D):
    BLK = 256
    nfull = V // BLK
    rem = V - nfull * BLK
    base_cnt = nfull // NW
    extra = nfull - base_cnt * NW
    iters = -(-(base_cnt + (1 if extra else 0)) // 2) * 2
    assert iters >= 4 and rem % 16 == 0 and BLK % 16 == 0

    mesh = plsc.VectorSubcoreMesh(core_axis_name="c", subcore_axis_name="s")

    @functools.partial(
        pl.kernel,
        mesh=mesh,
        compiler_params=pltpu.CompilerParams(
            needs_layout_passes=False, use_tc_tiling_on_sc=True),
        out_type=jax.ShapeDtypeStruct((V // 2, 2 * D), jnp.float32),
        scratch_types=(
            [pltpu.VMEM((D, BLK), jnp.float32) for _ in range(2)]
            + [pltpu.VMEM((BLK // 2, 2 * D + 1), jnp.float32)
               for _ in range(2)]
            + [pltpu.SemaphoreType.DMA for _ in range(4)]
        ),
    )
    def sc_transpose(wt_h, wtail_h, w2_h, *refs):
        inb = refs[0:2]
        outb = refs[2:4]
        semi = refs[4:6]
        semo = refs[6:8]

        wid = lax.axis_index("s") * NC + lax.axis_index("c")
        lanes = lax.iota(jnp.int32, 16)
        cnt = jnp.where(wid < extra, base_cnt + 1, base_cnt)
        bstart = wid * base_cnt + jnp.minimum(wid, extra)

        def blk_of(i):
            return bstart + jnp.minimum(i, cnt - 1)

        def fire_in(i, r):
            b = blk_of(i)
            for fb in range(D // 8):
                pltpu.async_copy(
                    wt_h.at[pl.ds(fb * 8, 8), pl.ds(b * BLK, BLK)],
                    inb[r].at[pl.ds(fb * 8, 8), :], semi[r])

        def wait_in(i, r):
            b = blk_of(i)
            for fb in range(D // 8):
                pltpu.make_async_copy(
                    wt_h.at[pl.ds(fb * 8, 8), pl.ds(b * BLK, BLK)],
                    inb[r].at[pl.ds(fb * 8, 8), :], semi[r]).wait()

        def fire_out(i, r):
            b = blk_of(i)
            pltpu.async_copy(
                outb[r].at[:, pl.ds(0, 2 * D)],
                w2_h.at[pl.ds(b * (BLK // 2), BLK // 2)], semo[r])

        def wait_out(i, r):
            b = blk_of(i)
            pltpu.make_async_copy(
                outb[r].at[:, pl.ds(0, 2 * D)],
                w2_h.at[pl.ds(b * (BLK // 2), BLK // 2)],
                semo[r]).wait()

        rowv = []
        colv = []
        for p in range(BLK // 16):
            ids = p * 16 + lanes
            rowv.append(ids >> 1)
            colv.append((ids & 1) * D)

        def transpose_blk(r):
            def fstep(f, _):
                for p in range(BLK // 16):
                    v = inb[r][f, pl.ds(p * 16, 16)]
                    plsc.store_scatter(outb[r], [rowv[p], colv[p] + f], v)
                return 0
            lax.fori_loop(0, D, fstep, 0)

        fire_in(0, 0)
        fire_in(1, 1)

        def step(k, _):
            for r in range(2):
                i = k * 2 + r

                @pl.when(i + 2 < iters)
                def _():
                    fire_in(i + 2, r)
                wait_in(i, r)

                @pl.when(i >= 2)
                def _():
                    wait_out(i - 2, r)
                transpose_blk(r)
                fire_out(i, r)
            return 0

        lax.fori_loop(0, iters // 2, step, 0)
        wait_out(iters - 2, 0)
        wait_out(iters - 1, 1)

        if rem:
            @pl.when(wid == NW - 1)
            def _():
                stage = outb[0].at[pl.ds(0, rem // 2), pl.ds(0, 2 * D)]
                pltpu.sync_copy(wtail_h, stage)
                pltpu.sync_copy(
                    stage, w2_h.at[pl.ds(nfull * (BLK // 2), rem // 2)])

    return sc_transpose


def kernel(pos_idxs, ys, neg_idxs, num_neg, W):
    B = pos_idxs.shape[1]
    NT = neg_idxs.shape[1]
    V, D = W.shape
    TOT = B + NT
    ppw = TOT // NW
    nch = ppw // C
    assert ppw % C == 0 and nch >= 2 * NBUF and D == 64 and V % 2 == 0
    scale = 1.0 / float(TOT)

    mesh = plsc.VectorSubcoreMesh(core_axis_name="c", subcore_axis_name="s")

    @functools.partial(
        pl.kernel,
        mesh=mesh,
        compiler_params=pltpu.CompilerParams(
            needs_layout_passes=False, use_tc_tiling_on_sc=True),
        out_type=jax.ShapeDtypeStruct((NW, 16), jnp.float32),
        scratch_types=(
            [pltpu.VMEM((ppw + 16,), jnp.int32) for _ in range(2)]
            + [pltpu.VMEM((ppw,), jnp.int32) for _ in range(2)]
            + [pltpu.VMEM((ppw,), jnp.float32)]
            + [pltpu.VMEM((C,), jnp.float32)]
            + [pltpu.VMEM((C, 2 * D), jnp.float32) for _ in range(2 * NBUF)]
            + [pltpu.VMEM((16,), jnp.float32)]
            + [pltpu.SemaphoreType.DMA for _ in range(2 * NBUF)]
        ),
    )
    def sc_loss(i0_h, i1_h, yse_h, w2_h, out_h, *refs):
        rawu, rawv, fetu, fetv, ysa, dots = refs[0:6]
        urows = refs[6:6 + NBUF]
        vrows = refs[6 + NBUF:6 + 2 * NBUF]
        accv = refs[6 + 2 * NBUF]
        semu = refs[7 + 2 * NBUF:7 + 2 * NBUF + NBUF]
        semv = refs[7 + 2 * NBUF + NBUF:7 + 2 * NBUF + 2 * NBUF]

        wid = lax.axis_index("s") * NC + lax.axis_index("c")
        tbase = wid * ppw
        lanes = lax.iota(jnp.int32, 16)

        pltpu.sync_copy(i0_h.at[pl.ds(tbase, ppw)], rawu.at[pl.ds(0, ppw)])
        pltpu.sync_copy(i1_h.at[pl.ds(tbase, ppw)], rawv.at[pl.ds(0, ppw)])
        pltpu.sync_copy(yse_h.at[pl.ds(tbase, ppw)], ysa)

        def shift_step(s, _):
            sl = pl.ds(s * 16, 16)
            fetu[sl] = rawu[sl] >> 1
            fetv[sl] = rawv[sl] >> 1
            return 0
        lax.fori_loop(0, ppw // 16, shift_step, 0)

        def fire(c, r):
            pltpu.async_copy(
                w2_h.at[fetu.at[pl.ds(c * C, C)]], urows[r], semu[r])
            pltpu.async_copy(
                w2_h.at[fetv.at[pl.ds(c * C, C)]], vrows[r], semv[r])

        def wait(c, r):
            pltpu.make_async_copy(
                w2_h.at[fetu.at[pl.ds(c * C, C)]], urows[r], semu[r]).wait()
            pltpu.make_async_copy(
                w2_h.at[fetv.at[pl.ds(c * C, C)]], vrows[r], semv[r]).wait()

        def compute(c, r, acc):
            def pair16(p16, _):
                u16 = (rawu[pl.ds(c * C + p16 * 16, 16)] & 1) * D
                v16 = (rawv[pl.ds(c * C + p16 * 16, 16)] & 1) * D
                dv = jnp.zeros((16,), jnp.float32)
                for q in range(16):
                    p = p16 * 16 + q
                    s = jnp.zeros((16,), jnp.float32)
                    for t in range(D // 16):
                        a = urows[r][p, pl.ds(u16[q] + t * 16, 16)]
                        b = vrows[r][p, pl.ds(v16[q] + t * 16, 16)]
                        s = s + a * b
                    dv = jnp.where(lanes == q, jnp.sum(s), dv)
                dots[pl.ds(p16 * 16, 16)] = dv
                return 0

            lax.fori_loop(0, C // 16, pair16, 0)

            def group(g, acc):
                avec = c * C + g * 16 + lanes
                dot = plsc.load_gather(dots, [g * 16 + lanes])
                z = jnp.where(tbase + avec < B, -dot, dot)
                t = jnp.exp(-jnp.abs(z))
                sp = jnp.maximum(z, 0.0) + _log_1to2(1.0 + t)
                yv = plsc.load_gather(ysa, [avec])
                alpha = _log_1to2(1.0 + _sqrt(yv)) + 1.0
                return acc + alpha * sp

            return lax.fori_loop(0, C // 16, group, acc)

        fire(0, 0)
        fire(1, 1)

        def step(k, acc):
            for r in range(NBUF):
                c = k * NBUF + r
                fire(c + 2, (r + 2) % NBUF)
                wait(c, r)
                acc = compute(c, r, acc)
            return acc

        assert nch % NBUF == 0
        acc = lax.fori_loop(0, nch // NBUF - 1, step,
                            jnp.zeros((16,), jnp.float32))
        c = nch - 3
        fire(nch - 1, (nch - 1) % NBUF)
        wait(c, c % NBUF)
        acc = compute(c, c % NBUF, acc)
        for c in range(nch - 2, nch):
            wait(c, c % NBUF)
            acc = compute(c, c % NBUF, acc)

        accv[...] = acc * scale
        pltpu.sync_copy(accv, out_h.at[wid])

    i0 = jnp.concatenate([pos_idxs[0], neg_idxs[0]])
    i1 = jnp.concatenate([pos_idxs[1], neg_idxs[1]])
    yse = jnp.concatenate([ys, jnp.zeros((NT,), jnp.float32)])
    rem = V % 256
    wtail = W[V - rem:].reshape(rem // 2, 2 * D) if rem else W[:8]
    w2 = _make_transpose(V, D)(W.T, wtail)
    partials = sc_loss(i0, i1, yse, w2)
    return jnp.sum(partials)
